# baseline (speedup 1.0000x reference)
import jax
import jax.numpy as jnp
from jax.experimental import pallas as pl
from jax.experimental.pallas import tpu as pltpu
from jax.experimental.pallas import tpu_sc as plsc

_B, _T, _D = 4, 8192, 1024
_H = 16
_HD = 64
_K = 128
_FF = 2816
_CT = 1024
_NW = 32
_RPW = (_B * _K) // _NW


def _sc_gather_body(hid_ref, cs_ref, fidx_ref, tidx_ref, sel_ref, css_ref,
                    idx_v, idx2_v, rows_v, cs_v, sem1, sem2):
    c = jax.lax.axis_index("c")
    s = jax.lax.axis_index("s")
    wid = s * 2 + c
    base = wid * _RPW
    pltpu.sync_copy(fidx_ref.at[pl.ds(base, _RPW)], idx_v)
    pltpu.async_copy(hid_ref.at[idx_v], rows_v, sem1).wait()
    pltpu.sync_copy(rows_v, sel_ref.at[pl.ds(base, _RPW)])
    pltpu.sync_copy(tidx_ref.at[pl.ds(base, _RPW)], idx2_v)
    pltpu.async_copy(cs_ref.at[idx2_v], cs_v, sem2).wait()
    pltpu.sync_copy(cs_v, css_ref.at[pl.ds(base, _RPW)])


def _decoder_body(sel_ref, cs_ref,
                  Wq, bq, Wk, bk, Wv, bv, Wo, ln1, ln2, Wg, Wu, Wd,
                  out_ref):
    sel = sel_ref[0]
    cosv = cs_ref[0, :, :_HD]
    sinv = cs_ref[0, :, _HD:]

    def rms(x, w):
        v = jnp.mean(x * x, axis=-1, keepdims=True)
        return x * jax.lax.rsqrt(v + 1e-6) * w

    def mm(x, w):
        return jax.lax.dot_general(
            x, w, (((1,), (0,)), ((), ())),
            preferred_element_type=jnp.float32)

    h = rms(sel, ln1[...])
    q = mm(h, Wq[...]) + bq[...]
    kk = mm(h, Wk[...]) + bk[...]
    v = mm(h, Wv[...]) + bv[...]

    def rope(x):
        x1 = x[:, :_HD // 2]
        x2 = x[:, _HD // 2:]
        rh = jnp.concatenate([-x2, x1], axis=1)
        return x * cosv + rh * sinv

    row_i = jax.lax.broadcasted_iota(jnp.int32, (_K, _K), 0)
    col_i = jax.lax.broadcasted_iota(jnp.int32, (_K, _K), 1)
    causal = col_i <= row_i
    neg = jnp.finfo(jnp.float32).min

    o_parts = []
    for hh in range(_H):
        sl = slice(hh * _HD, (hh + 1) * _HD)
        qh = rope(q[:, sl])
        kh = rope(kk[:, sl])
        vh = v[:, sl]
        s = jax.lax.dot_general(
            qh, kh, (((1,), (1,)), ((), ())),
            preferred_element_type=jnp.float32)
        s = s * (1.0 / (_HD ** 0.5))
        s = jnp.where(causal, s, neg)
        m = jnp.max(s, axis=-1, keepdims=True)
        p = jnp.exp(s - m)
        p = p / jnp.sum(p, axis=-1, keepdims=True)
        oh = jax.lax.dot_general(
            p, vh, (((1,), (0,)), ((), ())),
            preferred_element_type=jnp.float32)
        o_parts.append(oh)
    o = jnp.concatenate(o_parts, axis=1)

    h1 = sel + mm(o, Wo[...])
    h2 = rms(h1, ln2[...])
    ff2 = _FF // 2
    acc = h1
    for part in range(2):
        fsl = slice(part * ff2, (part + 1) * ff2)
        g = mm(h2, Wg[:, fsl])
        u = mm(h2, Wu[:, fsl])
        act = g * (1.0 / (1.0 + jnp.exp(-g))) * u
        acc = acc + mm(act, Wd[fsl, :])
    out_ref[0] = acc


def _copy_body(idx_ref, lo_ref, hi_ref, hid_ref, proc_ref, out_ref):
    b = pl.program_id(0)
    c = pl.program_id(1)
    out_ref[...] = hid_ref[...]
    base = c * _CT

    def sbody(k, carry):
        row = idx_ref[b, k] - base
        out_ref[0, pl.ds(row, 1), :] = proc_ref[0, pl.ds(k, 1), :]
        return carry

    jax.lax.fori_loop(lo_ref[b, c], hi_ref[b, c], sbody, 0)


def kernel(hidden_states, topk_indices, cos, sin, Wq, bq, Wk, bk, Wv, bv, Wo,
           ln1_w, ln2_w, Wgate, Wup, Wdown):
    B, T, D = hidden_states.shape
    K = topk_indices.shape[1]
    idx = topk_indices.astype(jnp.int32)

    hid_flat = hidden_states.reshape(B * T, D)
    cs_table = jnp.concatenate([cos[0], sin[0]], axis=-1)
    flat_idx = (idx + (jnp.arange(B, dtype=jnp.int32) * T)[:, None]).reshape(-1)
    tok_idx = idx.reshape(-1)

    mesh = plsc.VectorSubcoreMesh(core_axis_name="c", subcore_axis_name="s")
    sel_flat, css_flat = pl.kernel(
        _sc_gather_body,
        out_type=(jax.ShapeDtypeStruct((B * K, D), jnp.float32),
                  jax.ShapeDtypeStruct((B * K, 2 * _HD), jnp.float32)),
        mesh=mesh,
        scratch_types=[
            pltpu.VMEM((_RPW,), jnp.int32),
            pltpu.VMEM((_RPW,), jnp.int32),
            pltpu.VMEM((_RPW, _D), jnp.float32),
            pltpu.VMEM((_RPW, 2 * _HD), jnp.float32),
            pltpu.SemaphoreType.DMA,
            pltpu.SemaphoreType.DMA,
        ],
    )(hid_flat, cs_table, flat_idx, tok_idx)

    sel = sel_flat.reshape(B, K, D)
    css = css_flat.reshape(B, K, 2 * _HD)

    row = lambda x: x.reshape(1, -1)
    vm_full = lambda shape: pl.BlockSpec(shape, lambda b: (0,) * len(shape))

    processed = pl.pallas_call(
        _decoder_body,
        grid=(B,),
        in_specs=[
            pl.BlockSpec((1, K, D), lambda b: (b, 0, 0)),
            pl.BlockSpec((1, K, 2 * _HD), lambda b: (b, 0, 0)),
            vm_full((D, D)), vm_full((1, D)),
            vm_full((D, D)), vm_full((1, D)),
            vm_full((D, D)), vm_full((1, D)),
            vm_full((D, D)),
            vm_full((1, D)), vm_full((1, D)),
            vm_full((D, _FF)), vm_full((D, _FF)), vm_full((_FF, D)),
        ],
        out_specs=pl.BlockSpec((1, K, D), lambda b: (b, 0, 0)),
        out_shape=jax.ShapeDtypeStruct((B, K, D), jnp.float32),
    )(sel, css, Wq, row(bq), Wk, row(bk), Wv, row(bv), Wo,
      row(ln1_w), row(ln2_w), Wgate, Wup, Wdown)

    nch = T // _CT
    bounds = (jnp.arange(nch + 1, dtype=jnp.int32) * _CT)
    edges = jax.vmap(
        lambda r: jnp.searchsorted(r, bounds, side='left'))(idx)
    edges = edges.astype(jnp.int32)
    lo = edges[:, :-1]
    hi = edges[:, 1:]

    out = pl.pallas_call(
        _copy_body,
        grid_spec=pltpu.PrefetchScalarGridSpec(
            num_scalar_prefetch=3,
            grid=(B, nch),
            in_specs=[
                pl.BlockSpec((1, _CT, D), lambda b, c, i, l, h: (b, c, 0)),
                pl.BlockSpec((1, K, D), lambda b, c, i, l, h: (b, 0, 0)),
            ],
            out_specs=pl.BlockSpec((1, _CT, D), lambda b, c, i, l, h: (b, c, 0)),
        ),
        out_shape=jax.ShapeDtypeStruct((B, T, D), jnp.float32),
    )(idx, lo, hi, hidden_states, processed)
    return out
